# baseline (speedup 1.0000x reference)
import functools

import jax
import jax.numpy as jnp
from jax import lax
from jax.experimental import pallas as pl
from jax.experimental.pallas import tpu as pltpu
from jax.experimental.pallas import tpu_sc as plsc

N = 10000
E = 320000
D = 128
G = 128

NC = 2
NS = 16
NW = NC * NS
CH = 76
NCH = 132
EPW = NCH * CH
EPAD = NW * EPW
ZROW = N
RPT = 624
RPT_LAST = N - 15 * RPT

_sc_mesh = plsc.VectorSubcoreMesh(core_axis_name="c", subcore_axis_name="s")


@functools.partial(
    pl.kernel,
    out_type=(
        jax.ShapeDtypeStruct((N, D), jnp.float32),
        jax.ShapeDtypeStruct((N, D), jnp.float32),
    ),
    mesh=_sc_mesh,
    scratch_types=[
        pltpu.VMEM_SHARED((N, D), jnp.float32),
        pltpu.VMEM((NCH, CH), jnp.int32),
        pltpu.VMEM((NCH, CH), jnp.int32),
        pltpu.VMEM((CH, D), jnp.float32),
    ],
)
def _sc_agg(x_hbm, src_hbm, dst_hbm, out0, out1, agg_sh, di, si, r0):
    c = lax.axis_index("c")
    s = lax.axis_index("s")
    wid = c * NS + s

    def zrow(i, carry):
        r0[i, pl.ds(0, 16)] = jnp.zeros((16,), jnp.float32)
        r0[i, pl.ds(16, 16)] = jnp.zeros((16,), jnp.float32)
        r0[i, pl.ds(32, 16)] = jnp.zeros((16,), jnp.float32)
        r0[i, pl.ds(48, 16)] = jnp.zeros((16,), jnp.float32)
        r0[i, pl.ds(64, 16)] = jnp.zeros((16,), jnp.float32)
        r0[i, pl.ds(80, 16)] = jnp.zeros((16,), jnp.float32)
        r0[i, pl.ds(96, 16)] = jnp.zeros((16,), jnp.float32)
        r0[i, pl.ds(112, 16)] = jnp.zeros((16,), jnp.float32)
        return carry

    lax.fori_loop(0, CH, zrow, 0)
    row0 = pl.multiple_of(s * RPT, 8)

    @pl.when(s < NS - 1)
    def _():
        for k in range(RPT // CH):
            pltpu.sync_copy(r0, agg_sh.at[pl.ds(row0 + k * CH, CH)])
        if RPT % CH:
            pltpu.sync_copy(r0.at[pl.ds(0, RPT % CH)],
                            agg_sh.at[pl.ds(row0 + RPT - RPT % CH, RPT % CH)])

    @pl.when(s == NS - 1)
    def _():
        for k in range(RPT_LAST // CH):
            pltpu.sync_copy(r0, agg_sh.at[pl.ds(15 * RPT + k * CH, CH)])
        if RPT_LAST % CH:
            pltpu.sync_copy(
                r0.at[pl.ds(0, RPT_LAST % CH)],
                agg_sh.at[pl.ds(15 * RPT + RPT_LAST - RPT_LAST % CH,
                                RPT_LAST % CH)])

    plsc.subcore_barrier()

    pltpu.sync_copy(dst_hbm.at[wid], di)
    pltpu.sync_copy(src_hbm.at[wid], si)

    def body(ci, carry):
        pltpu.sync_copy(x_hbm.at[si.at[ci]], r0)
        pltpu.sync_copy(r0, agg_sh.at[di.at[ci]], add=True)
        return carry

    lax.fori_loop(0, NCH, body, 0)
    plsc.subcore_barrier()

    @pl.when((c == 0) & (s < NS - 1))
    def _():
        pltpu.sync_copy(agg_sh.at[pl.ds(row0, RPT)], out0.at[pl.ds(row0, RPT)])

    @pl.when((c == 0) & (s == NS - 1))
    def _():
        pltpu.sync_copy(agg_sh.at[pl.ds(15 * RPT, RPT_LAST)],
                        out0.at[pl.ds(15 * RPT, RPT_LAST)])

    @pl.when((c == 1) & (s < NS - 1))
    def _():
        pltpu.sync_copy(agg_sh.at[pl.ds(row0, RPT)], out1.at[pl.ds(row0, RPT)])

    @pl.when((c == 1) & (s == NS - 1))
    def _():
        pltpu.sync_copy(agg_sh.at[pl.ds(15 * RPT, RPT_LAST)],
                        out1.at[pl.ds(15 * RPT, RPT_LAST)])


BM = 1000
NBLK = N // BM


def _tc_body(x_ref, a0_ref, a1_ref, batch_ref, w_ref, b_ref, wr_ref, br_ref,
             out_ref, sums_ref, counts_ref):
    i = pl.program_id(0)

    @pl.when(i == 0)
    def _():
        sums_ref[...] = jnp.zeros_like(sums_ref)
        counts_ref[...] = jnp.zeros_like(counts_ref)

    t = x_ref[...] + a0_ref[...] + a1_ref[...]
    h = jnp.maximum(
        jnp.dot(t, w_ref[...], preferred_element_type=jnp.float32) + b_ref[...],
        0.0,
    )
    gids = lax.broadcasted_iota(jnp.int32, (1, G), 1)
    onehot = (batch_ref[...] == gids).astype(jnp.float32)
    sums_ref[...] += lax.dot_general(
        onehot, h, (((0,), (0,)), ((), ())),
        preferred_element_type=jnp.float32)
    counts_ref[...] += lax.dot_general(
        onehot, jnp.ones((BM, D), jnp.float32), (((0,), (0,)), ((), ())),
        preferred_element_type=jnp.float32)

    @pl.when(i == NBLK - 1)
    def _():
        pool = sums_ref[...] / jnp.maximum(counts_ref[...], 1.0)
        out_ref[...] = (
            jnp.dot(pool, wr_ref[...], preferred_element_type=jnp.float32)
            + br_ref[...]
        )


def kernel(x, edge_index, batch, W, b, Wr, br):
    src = edge_index[0].astype(jnp.int32)
    dst = edge_index[1].astype(jnp.int32)
    pad = EPAD - E
    src3 = jnp.concatenate([src, jnp.full((pad,), ZROW, jnp.int32)]).reshape(
        NW, NCH, CH)
    dst3 = jnp.concatenate([dst, jnp.zeros((pad,), jnp.int32)]).reshape(
        NW, NCH, CH)
    x_pad = jnp.concatenate([x, jnp.zeros((8, D), jnp.float32)])

    agg0, agg1 = _sc_agg(x_pad, src3, dst3)

    batch2d = batch.astype(jnp.int32).reshape(N, 1)
    b2d = b.reshape(1, D)
    br2d = br.reshape(1, 1)

    out = pl.pallas_call(
        _tc_body,
        grid=(NBLK,),
        in_specs=[
            pl.BlockSpec((BM, D), lambda i: (i, 0)),
            pl.BlockSpec((BM, D), lambda i: (i, 0)),
            pl.BlockSpec((BM, D), lambda i: (i, 0)),
            pl.BlockSpec((BM, 1), lambda i: (i, 0)),
            pl.BlockSpec((D, D), lambda i: (0, 0)),
            pl.BlockSpec((1, D), lambda i: (0, 0)),
            pl.BlockSpec((D, 1), lambda i: (0, 0)),
            pl.BlockSpec((1, 1), lambda i: (0, 0)),
        ],
        out_specs=pl.BlockSpec((G, 1), lambda i: (0, 0)),
        out_shape=jax.ShapeDtypeStruct((G, 1), jnp.float32),
        scratch_shapes=[
            pltpu.VMEM((G, D), jnp.float32),
            pltpu.VMEM((G, D), jnp.float32),
        ],
        compiler_params=pltpu.CompilerParams(
            dimension_semantics=("arbitrary",)),
    )(x, agg0, agg1, batch2d, W, b2d, Wr, br2d)

    return out.reshape(G)


# speedup vs baseline: 1.1690x; 1.1690x over previous
import functools

import jax
import jax.numpy as jnp
from jax import lax
from jax.experimental import pallas as pl
from jax.experimental.pallas import tpu as pltpu
from jax.experimental.pallas import tpu_sc as plsc

N = 10000
E = 320000
D = 128
G = 128

NC = 2
NS = 16
NW = NC * NS
CH = 80
NCH = 125
EPW = NCH * CH
EPAD = NW * EPW
ZROW = N
RPT = 624
RPT_LAST = N - 15 * RPT

_sc_mesh = plsc.VectorSubcoreMesh(core_axis_name="c", subcore_axis_name="s")


@functools.partial(
    pl.kernel,
    out_type=(
        jax.ShapeDtypeStruct((N, D), jnp.float32),
        jax.ShapeDtypeStruct((N, D), jnp.float32),
    ),
    mesh=_sc_mesh,
    scratch_types=[
        pltpu.VMEM_SHARED((N, D), jnp.float32),
        pltpu.VMEM((NCH, CH), jnp.int32),
        pltpu.VMEM((NCH, CH), jnp.int32),
        pltpu.VMEM((CH, D), jnp.float32),
    ],
)
def _sc_agg(x_hbm, src_hbm, dst_hbm, out0, out1, agg_sh, di, si, r0):
    c = lax.axis_index("c")
    s = lax.axis_index("s")
    wid = c * NS + s

    def zrow(i, carry):
        r0[i, pl.ds(0, 16)] = jnp.zeros((16,), jnp.float32)
        r0[i, pl.ds(16, 16)] = jnp.zeros((16,), jnp.float32)
        r0[i, pl.ds(32, 16)] = jnp.zeros((16,), jnp.float32)
        r0[i, pl.ds(48, 16)] = jnp.zeros((16,), jnp.float32)
        r0[i, pl.ds(64, 16)] = jnp.zeros((16,), jnp.float32)
        r0[i, pl.ds(80, 16)] = jnp.zeros((16,), jnp.float32)
        r0[i, pl.ds(96, 16)] = jnp.zeros((16,), jnp.float32)
        r0[i, pl.ds(112, 16)] = jnp.zeros((16,), jnp.float32)
        return carry

    lax.fori_loop(0, CH, zrow, 0)
    row0 = pl.multiple_of(s * RPT, 8)

    @pl.when(s < NS - 1)
    def _():
        for k in range(RPT // CH):
            pltpu.sync_copy(r0, agg_sh.at[pl.ds(row0 + k * CH, CH)])
        if RPT % CH:
            pltpu.sync_copy(r0.at[pl.ds(0, RPT % CH)],
                            agg_sh.at[pl.ds(row0 + RPT - RPT % CH, RPT % CH)])

    @pl.when(s == NS - 1)
    def _():
        for k in range(RPT_LAST // CH):
            pltpu.sync_copy(r0, agg_sh.at[pl.ds(15 * RPT + k * CH, CH)])
        if RPT_LAST % CH:
            pltpu.sync_copy(
                r0.at[pl.ds(0, RPT_LAST % CH)],
                agg_sh.at[pl.ds(15 * RPT + RPT_LAST - RPT_LAST % CH,
                                RPT_LAST % CH)])

    plsc.subcore_barrier()

    pltpu.sync_copy(dst_hbm.at[wid], di)
    pltpu.sync_copy(src_hbm.at[wid], si)

    def body(ci, carry):
        pltpu.sync_copy(x_hbm.at[si.at[ci]], r0)
        pltpu.sync_copy(r0, agg_sh.at[di.at[ci]], add=True)
        return carry

    lax.fori_loop(0, NCH, body, 0)
    plsc.subcore_barrier()

    @pl.when((c == 0) & (s < NS - 1))
    def _():
        pltpu.sync_copy(agg_sh.at[pl.ds(row0, RPT)], out0.at[pl.ds(row0, RPT)])

    @pl.when((c == 0) & (s == NS - 1))
    def _():
        pltpu.sync_copy(agg_sh.at[pl.ds(15 * RPT, RPT_LAST)],
                        out0.at[pl.ds(15 * RPT, RPT_LAST)])

    @pl.when((c == 1) & (s < NS - 1))
    def _():
        pltpu.sync_copy(agg_sh.at[pl.ds(row0, RPT)], out1.at[pl.ds(row0, RPT)])

    @pl.when((c == 1) & (s == NS - 1))
    def _():
        pltpu.sync_copy(agg_sh.at[pl.ds(15 * RPT, RPT_LAST)],
                        out1.at[pl.ds(15 * RPT, RPT_LAST)])


BM = 2000
NBLK = N // BM


def _tc_body(x_ref, a0_ref, a1_ref, batch_ref, w_ref, b_ref, wr_ref, br_ref,
             out_ref, sums_ref, counts_ref):
    i = pl.program_id(0)

    @pl.when(i == 0)
    def _():
        sums_ref[...] = jnp.zeros_like(sums_ref)
        counts_ref[...] = jnp.zeros_like(counts_ref)

    t = x_ref[...] + a0_ref[...] + a1_ref[...]
    h = jnp.maximum(
        jnp.dot(t, w_ref[...], preferred_element_type=jnp.float32) + b_ref[...],
        0.0,
    )
    gids = lax.broadcasted_iota(jnp.int32, (1, G), 1)
    onehot = (batch_ref[...] == gids).astype(jnp.float32)
    sums_ref[...] += lax.dot_general(
        onehot, h, (((0,), (0,)), ((), ())),
        preferred_element_type=jnp.float32)
    counts_ref[...] += jnp.sum(onehot, axis=0, keepdims=True)

    @pl.when(i == NBLK - 1)
    def _():
        pool = sums_ref[...] / jnp.maximum(counts_ref[...], 1.0).T
        out_ref[...] = (
            jnp.dot(pool, wr_ref[...], preferred_element_type=jnp.float32)
            + br_ref[...]
        )


def kernel(x, edge_index, batch, W, b, Wr, br):
    src = edge_index[0].astype(jnp.int32)
    dst = edge_index[1].astype(jnp.int32)
    pad = EPAD - E
    src3 = jnp.concatenate([src, jnp.full((pad,), ZROW, jnp.int32)]).reshape(
        NW, NCH, CH)
    dst3 = jnp.concatenate([dst, jnp.zeros((pad,), jnp.int32)]).reshape(
        NW, NCH, CH)
    x_pad = jnp.concatenate([x, jnp.zeros((8, D), jnp.float32)])

    agg0, agg1 = _sc_agg(x_pad, src3, dst3)

    batch2d = batch.astype(jnp.int32).reshape(N, 1)
    b2d = b.reshape(1, D)
    br2d = br.reshape(1, 1)

    out = pl.pallas_call(
        _tc_body,
        grid=(NBLK,),
        in_specs=[
            pl.BlockSpec((BM, D), lambda i: (i, 0)),
            pl.BlockSpec((BM, D), lambda i: (i, 0)),
            pl.BlockSpec((BM, D), lambda i: (i, 0)),
            pl.BlockSpec((BM, 1), lambda i: (i, 0)),
            pl.BlockSpec((D, D), lambda i: (0, 0)),
            pl.BlockSpec((1, D), lambda i: (0, 0)),
            pl.BlockSpec((D, 1), lambda i: (0, 0)),
            pl.BlockSpec((1, 1), lambda i: (0, 0)),
        ],
        out_specs=pl.BlockSpec((G, 1), lambda i: (0, 0)),
        out_shape=jax.ShapeDtypeStruct((G, 1), jnp.float32),
        scratch_shapes=[
            pltpu.VMEM((G, D), jnp.float32),
            pltpu.VMEM((1, G), jnp.float32),
        ],
        compiler_params=pltpu.CompilerParams(
            dimension_semantics=("arbitrary",)),
    )(x, agg0, agg1, batch2d, W, b2d, Wr, br2d)

    return out.reshape(G)
